# baseline (speedup 1.0000x reference)
import functools

import jax
import jax.numpy as jnp
from jax import lax
from jax.experimental import pallas as pl
from jax.experimental.pallas import tpu as pltpu
from jax.experimental.pallas import tpu_sc as plsc

HIDDEN = 64
CHUNK = 128
NBUF = 4


@functools.lru_cache(maxsize=None)
def _make_gather(B, V):
    info = plsc.get_sparse_core_info()
    nc, ns = info.num_cores, info.num_subcores
    nw = nc * ns
    rows_per_w = B // nw
    ng = rows_per_w // CHUNK
    assert rows_per_w % CHUNK == 0 and ng % NBUF == 0

    mesh = plsc.VectorSubcoreMesh(core_axis_name="c", subcore_axis_name="s")

    @functools.partial(
        pl.kernel,
        mesh=mesh,
        compiler_params=pltpu.CompilerParams(use_tc_tiling_on_sc=False),
        out_type=jax.ShapeDtypeStruct((B, HIDDEN), jnp.float32),
        scratch_types=(
            [pltpu.VMEM((rows_per_w,), jnp.int32)]
            + [pltpu.VMEM((CHUNK, HIDDEN), jnp.float32) for _ in range(NBUF)]
            + [pltpu.SemaphoreType.DMA for _ in range(NBUF)]
        ),
    )
    def gather_kernel(idx_hbm, table_hbm, out_hbm, idx_v, *rest):
        rows = rest[:NBUF]
        sems = rest[NBUF:]
        wid = lax.axis_index("s") * nc + lax.axis_index("c")
        out_base = wid * rows_per_w

        pltpu.sync_copy(idx_hbm.at[pl.ds(out_base, rows_per_w)], idx_v)

        for b in range(NBUF):
            pltpu.async_copy(
                table_hbm.at[idx_v.at[pl.ds(b * CHUNK, CHUNK)]], rows[b], sems[b]
            )

        def outer(i, carry):
            for b in range(NBUF):
                g = i * NBUF + b
                pltpu.make_async_copy(
                    table_hbm.at[idx_v.at[pl.ds(g * CHUNK, CHUNK)]],
                    rows[b],
                    sems[b],
                ).wait()
                pltpu.sync_copy(
                    rows[b], out_hbm.at[pl.ds(out_base + g * CHUNK, CHUNK)]
                )

                @pl.when(g + NBUF < ng)
                def _():
                    pltpu.async_copy(
                        table_hbm.at[idx_v.at[pl.ds((g + NBUF) * CHUNK, CHUNK)]],
                        rows[b],
                        sems[b],
                    )

            return carry

        lax.fori_loop(0, ng // NBUF, outer, 0)

    return gather_kernel


def kernel(input_var, emb_weight):
    shape = input_var.shape
    idx = input_var.reshape(-1).astype(jnp.int32)
    out = _make_gather(idx.shape[0], emb_weight.shape[0])(idx, emb_weight)
    return out.reshape(*shape, HIDDEN)
